# baseline (speedup 1.0000x reference)
import functools

import jax
import jax.numpy as jnp
from jax import lax
from jax.experimental import pallas as pl
from jax.experimental.pallas import tpu as pltpu
from jax.experimental.pallas import tpu_sc as plsc

_HIDDEN = 64
_LANES = 16

_NC = 2
_NS = 16
_NW = _NC * _NS
_BLK = 128


def _body(nbatch, seq, half_hbm, par_hbm, tablep_hbm, out_hbm,
          half_v, par_v, rows0, rows1, ob0, ob1, gs0, gs1, ws0, ws1):
    wid = lax.axis_index("s") * _NC + lax.axis_index("c")
    col0 = wid * _BLK

    rows = (rows0, rows1)
    obuf = (ob0, ob1)
    gsem = (gs0, gs1)
    wsem = (ws0, ws1)

    pltpu.sync_copy(half_hbm.at[:, pl.ds(col0, _BLK)], half_v)
    pltpu.sync_copy(par_hbm.at[:, pl.ds(col0, _BLK)], par_v)

    iotas = [lax.broadcasted_iota(jnp.int32, (_LANES,), 0) + k * _LANES
             for k in range(_HIDDEN // _LANES)]

    def out_slice(s):
        return out_hbm.at[s, :, pl.ds(col0, _BLK)]

    def gather(s, r):
        return pltpu.async_copy(
            tablep_hbm.at[half_v.at[s]], rows[r], gsem[r])

    def transpose_block(s, rv, ob):
        def grp_body(m, carry):
            rowidx = iotas[0] + m * _LANES
            parbase = par_v[s, pl.ds(m * _LANES, _LANES)] * _HIDDEN
            def h_body(h, c2):
                v = plsc.load_gather(rv, [rowidx, parbase + h])
                ob[h, pl.ds(m * _LANES, _LANES)] = v
                return c2
            lax.fori_loop(0, _HIDDEN, h_body, 0, unroll=8)
            return carry
        lax.fori_loop(0, _BLK // _LANES, grp_body, 0)

    gather(0, 0)

    def step(j, carry):
        for b in range(2):
            s = 2 * j + b
            nb = 1 - b
            @pl.when(s + 1 < seq)
            def _fire():
                gather(s + 1, nb)
            pltpu.make_async_copy(
                tablep_hbm.at[half_v.at[s]], rows[b], gsem[b]).wait()
            @pl.when(s >= 2)
            def _drain():
                pltpu.make_async_copy(obuf[b], out_slice(s - 2), wsem[b]).wait()
            transpose_block(s, rows[b], obuf[b])
            pltpu.async_copy(obuf[b], out_slice(s), wsem[b])
        return carry

    lax.fori_loop(0, seq // 2, step, 0)

    pltpu.make_async_copy(ob0, out_slice(seq - 2), ws0).wait()
    pltpu.make_async_copy(ob1, out_slice(seq - 1), ws1).wait()


def kernel(pos_embedding_matrix, position_labels):
    b, s = position_labels.shape
    assert b % (_NW * _BLK) == 0 or b == _NW * _BLK
    assert s % 2 == 0

    lt = position_labels.astype(jnp.int32).T
    half = lt >> 1
    par = lt & 1
    tablep = pos_embedding_matrix.reshape(-1, 2 * _HIDDEN)

    mesh = plsc.VectorSubcoreMesh(core_axis_name="c", subcore_axis_name="s")
    run = pl.kernel(
        functools.partial(_body, b, s),
        mesh=mesh,
        compiler_params=pltpu.CompilerParams(
            use_tc_tiling_on_sc=True, needs_layout_passes=False),
        out_type=jax.ShapeDtypeStruct((s, _HIDDEN, b), jnp.float32),
        scratch_types=[
            pltpu.VMEM((s, _BLK), jnp.int32),
            pltpu.VMEM((s, _BLK), jnp.int32),
            pltpu.VMEM((_BLK, 2 * _HIDDEN), jnp.float32),
            pltpu.VMEM((_BLK, 2 * _HIDDEN), jnp.float32),
            pltpu.VMEM((_HIDDEN, _BLK), jnp.float32),
            pltpu.VMEM((_HIDDEN, _BLK), jnp.float32),
            pltpu.SemaphoreType.DMA,
            pltpu.SemaphoreType.DMA,
            pltpu.SemaphoreType.DMA,
            pltpu.SemaphoreType.DMA,
        ],
    )
    raw = run(half, par, tablep)
    return jnp.transpose(raw, (2, 0, 1))


# speedup vs baseline: 1.8818x; 1.8818x over previous
import functools

import jax
import jax.numpy as jnp
from jax import lax
from jax.experimental import pallas as pl
from jax.experimental.pallas import tpu as pltpu
from jax.experimental.pallas import tpu_sc as plsc

_HIDDEN = 64
_LANES = 16

_NC = 2
_NS = 16
_NW = _NC * _NS
_BLK = 128


def _body(nbatch, seq, half_hbm, par_hbm, tablep_hbm, out_hbm,
          half_v, par_v, rows0, rows1, ob0, ob1, gs0, gs1, ws0, ws1):
    wid = lax.axis_index("s") * _NC + lax.axis_index("c")
    col0 = wid * _BLK

    rows = (rows0, rows1)
    obuf = (ob0, ob1)
    gsem = (gs0, gs1)
    wsem = (ws0, ws1)

    pltpu.sync_copy(half_hbm.at[:, pl.ds(col0, _BLK)], half_v)
    pltpu.sync_copy(par_hbm.at[:, pl.ds(col0, _BLK)], par_v)

    iotas = [lax.broadcasted_iota(jnp.int32, (_LANES,), 0) + k * _LANES
             for k in range(_HIDDEN // _LANES)]

    def out_slice(s):
        return out_hbm.at[s, :, pl.ds(col0, _BLK)]

    def gather(s, r):
        return pltpu.async_copy(
            tablep_hbm.at[half_v.at[s]], rows[r], gsem[r])

    def transpose_block(s, rv, ob):
        ngrp = _BLK // _LANES
        rowidx = [iotas[0] + m * _LANES for m in range(ngrp)]
        parbase = [par_v[s, pl.ds(m * _LANES, _LANES)] * _HIDDEN
                   for m in range(ngrp)]

        @plsc.parallel_loop(0, _HIDDEN, unroll=8)
        def h_body(h):
            for m in range(ngrp):
                v = plsc.load_gather(rv, [rowidx[m], parbase[m] + h])
                ob[h, pl.ds(m * _LANES, _LANES)] = v

    gather(0, 0)

    def step(j, carry):
        for b in range(2):
            s = 2 * j + b
            nb = 1 - b
            @pl.when(s + 1 < seq)
            def _fire():
                gather(s + 1, nb)
            pltpu.make_async_copy(
                tablep_hbm.at[half_v.at[s]], rows[b], gsem[b]).wait()
            @pl.when(s >= 2)
            def _drain():
                pltpu.make_async_copy(obuf[b], out_slice(s - 2), wsem[b]).wait()
            transpose_block(s, rows[b], obuf[b])
            pltpu.async_copy(obuf[b], out_slice(s), wsem[b])
        return carry

    lax.fori_loop(0, seq // 2, step, 0)

    pltpu.make_async_copy(ob0, out_slice(seq - 2), ws0).wait()
    pltpu.make_async_copy(ob1, out_slice(seq - 1), ws1).wait()


def kernel(pos_embedding_matrix, position_labels):
    b, s = position_labels.shape
    assert b % (_NW * _BLK) == 0 or b == _NW * _BLK
    assert s % 2 == 0

    lt = position_labels.astype(jnp.int32).T
    half = lt >> 1
    par = lt & 1
    tablep = pos_embedding_matrix.reshape(-1, 2 * _HIDDEN)

    mesh = plsc.VectorSubcoreMesh(core_axis_name="c", subcore_axis_name="s")
    run = pl.kernel(
        functools.partial(_body, b, s),
        mesh=mesh,
        compiler_params=pltpu.CompilerParams(
            use_tc_tiling_on_sc=True, needs_layout_passes=False),
        out_type=jax.ShapeDtypeStruct((s, _HIDDEN, b), jnp.float32),
        scratch_types=[
            pltpu.VMEM((s, _BLK), jnp.int32),
            pltpu.VMEM((s, _BLK), jnp.int32),
            pltpu.VMEM((_BLK, 2 * _HIDDEN), jnp.float32),
            pltpu.VMEM((_BLK, 2 * _HIDDEN), jnp.float32),
            pltpu.VMEM((_HIDDEN, _BLK), jnp.float32),
            pltpu.VMEM((_HIDDEN, _BLK), jnp.float32),
            pltpu.SemaphoreType.DMA,
            pltpu.SemaphoreType.DMA,
            pltpu.SemaphoreType.DMA,
            pltpu.SemaphoreType.DMA,
        ],
    )
    raw = run(half, par, tablep)
    return jnp.transpose(raw, (2, 0, 1))
